# baseline (speedup 1.0000x reference)
import functools

import jax
import jax.numpy as jnp
from jax import lax
from jax.experimental import pallas as pl
from jax.experimental.pallas import tpu as pltpu
from jax.experimental.pallas import tpu_sc as plsc

F32 = jnp.float32
ND, NPRO, NTOT = 3584, 512, 4096
EMB, H1, H2 = 128, 64, 32
V, L, KW, OC, TT = 26, 1000, 8, 32, 121

NC_SC, NS_SC = 2, 16
NW = NC_SC * NS_SC
G = EMB // 16
LP = 1024
CT = LP // NW
SB = 16
NSUB = CT // SB


def _prep_body(emb_ref, wfc_ref, w5_ref, cbrow_ref, bfc_ref,
               x_ref, w1_ref, b1_ref, w2_ref, b2_ref, w3_ref, b3_ref, wg1_ref,
               u_ref, bias_ref, y1d_ref, t_ref):
    for k in range(KW):
        ek = emb_ref[:, k:k + TT]
        for o in range(OC):
            wo = wfc_ref[pl.ds(o * TT, TT), :]
            t_ref[k, o] = jnp.dot(ek, wo, preferred_element_type=F32)
    for v in range(V):
        tv = t_ref[:, :, v, :].reshape(KW * OC, EMB)
        uv = jnp.dot(w5_ref[...], tv, preferred_element_type=F32)
        u_ref[v, pl.ds(0, L), :] = uv.astype(jnp.bfloat16)
        u_ref[v, pl.ds(L, LP - L), :] = jnp.zeros((LP - L, EMB), jnp.bfloat16)
    bias_ref[...] = (jnp.dot(cbrow_ref[...], wfc_ref[...], preferred_element_type=F32)
                     + bfc_ref[...])
    h = jnp.maximum(jnp.dot(x_ref[...], w1_ref[...], preferred_element_type=F32)
                    + b1_ref[...], 0.0)
    h = jnp.maximum(jnp.dot(h, w2_ref[...], preferred_element_type=F32)
                    + b2_ref[...], 0.0)
    h = jnp.maximum(jnp.dot(h, w3_ref[...], preferred_element_type=F32)
                    + b3_ref[...], 0.0)
    y1d_ref[...] = jnp.dot(h, wg1_ref[...], preferred_element_type=F32)


def _aggr_body(adj_ref, y_ref, o_ref):
    o_ref[...] = jnp.dot(adj_ref[...], y_ref[...], preferred_element_type=F32)


def _aggrb_y23_body(pa_ref, adj_ref, praw_ref, bias_ref, wg1_ref, w23_ref, o_ref):
    y1p = jnp.dot(praw_ref[0] + praw_ref[1] + bias_ref[...], wg1_ref[...],
                  preferred_element_type=F32)
    h = jnp.maximum(pa_ref[...] + jnp.dot(adj_ref[...], y1p,
                                          preferred_element_type=F32), 0.0)
    o_ref[...] = jnp.dot(h, w23_ref[...], preferred_element_type=F32)


def _aggr2_body(adj_ref, y_ref, mu_ref, lv_ref):
    r = jnp.dot(adj_ref[...], y_ref[...], preferred_element_type=F32)
    mu_ref[...] = r[:, :H2]
    lv_ref[...] = r[:, H2:]


def _dec_body(a_ref, b_ref, o_ref):
    o_ref[...] = lax.dot_general(a_ref[...], b_ref[...],
                                 (((1,), (1,)), ((), ())),
                                 preferred_element_type=F32)


def _sc_body(u_hbm, idx_hbm, out_hbm, u_sub, idx_v, acc_v, idxrow_v, shared):
    cid = lax.axis_index("c")
    sid = lax.axis_index("s")
    wid = sid * NC_SC + cid
    c0 = wid * CT
    pltpu.sync_copy(idx_hbm.at[pl.ds(c0, CT)], idx_v)

    def zbody(n16, carry):
        for g in range(G):
            acc_v[n16, pl.ds(g * 16, 16)] = jnp.zeros((16,), F32)
        return carry

    lax.fori_loop(0, NPRO, zbody, 0)

    def sbody(sub, carry):
        off = wid * (CT // 2) + sub * (SB // 2)
        pltpu.sync_copy(u_hbm.at[:, pl.ds(off, SB // 2), :], u_sub)

        def nbody(nb, carry2):
            n0 = nb * 16
            iv = [idx_v[sub * SB + c, pl.ds(n0, 16)] for c in range(SB)]
            for j in range(16):
                n = n0 + j
                acc = [acc_v[n, pl.ds(g * 16, 16)] for g in range(G)]
                for c in range(SB):
                    v = iv[c][j]
                    base = (c % 2) * (EMB // 2)
                    for b4 in range(4):
                        x = u_sub[v, c // 2, pl.ds(base + b4 * 16, 16)]
                        lo = lax.bitcast_convert_type(lax.shift_left(x, 16), F32)
                        hi = lax.bitcast_convert_type(
                            jnp.bitwise_and(x, jnp.int32(-65536)), F32)
                        acc[2 * b4] = acc[2 * b4] + lo
                        acc[2 * b4 + 1] = acc[2 * b4 + 1] + hi
                for g in range(G):
                    acc_v[n, pl.ds(g * 16, 16)] = acc[g]
            return carry2

        lax.fori_loop(0, NPRO // 16, nbody, 0)
        return carry

    lax.fori_loop(0, NSUB, sbody, 0)

    for blk in range(NPRO // EMB):
        for g in range(G):
            idxrow_v[blk, pl.ds(g * 16, 16)] = (lax.iota(jnp.int32, 16)
                                                + (blk * EMB + g * 16))

    @pl.when(sid == 0)
    def _():
        pltpu.sync_copy(acc_v, shared)
    plsc.subcore_barrier()

    @pl.when(sid != 0)
    def _():
        for blk in range(NPRO // EMB):
            pltpu.sync_copy(acc_v.at[pl.ds(blk * EMB, EMB)],
                            shared.at[idxrow_v.at[blk]], add=True)
    plsc.subcore_barrier()

    @pl.when(sid == 0)
    def _():
        pltpu.sync_copy(shared, out_hbm.at[cid])


_sc_gather = functools.partial(
    pl.kernel,
    out_type=jax.ShapeDtypeStruct((NC_SC, NPRO, EMB), F32),
    mesh=plsc.VectorSubcoreMesh(core_axis_name="c", subcore_axis_name="s"),
    scratch_types=[
        pltpu.VMEM((V, SB // 2, EMB), jnp.int32),
        pltpu.VMEM((CT, NPRO), jnp.int32),
        pltpu.VMEM((NPRO, EMB), F32),
        pltpu.VMEM((NPRO // EMB, EMB), jnp.int32),
        pltpu.VMEM_SHARED((NPRO, EMB), F32),
    ],
)(_sc_body)


def kernel(drug_x, pro_x, net_adj, W1, b1, W2, b2, W3, b3, emb_table,
           conv_w, conv_b, Wfc, bfc, Wg1, Wg2, Wg3):
    w5 = conv_w.transpose(1, 2, 0).reshape(L, KW * OC)
    cbrow = jnp.repeat(conv_b, TT)[None, :]
    b1r, b2r, b3r, bfcr = b1[None, :], b2[None, :], b3[None, :], bfc[None, :]

    u, bias, y1d = pl.pallas_call(
        _prep_body,
        out_shape=(jax.ShapeDtypeStruct((V, LP, EMB), jnp.bfloat16),
                   jax.ShapeDtypeStruct((1, EMB), F32),
                   jax.ShapeDtypeStruct((ND, H1), F32)),
        scratch_shapes=[pltpu.VMEM((KW, OC, V, EMB), F32)],
    )(emb_table, Wfc, w5, cbrow, bfcr, drug_x, W1, b1r, W2, b2r, W3, b3r, Wg1)

    u_i32 = lax.bitcast_convert_type(
        u.reshape(V, LP, EMB // 2, 2), jnp.int32).reshape(V, LP // 2, EMB)

    idx_t = jnp.pad(pro_x.transpose(), ((0, LP - L), (0, 0)))

    pro_raw = _sc_gather(u_i32, idx_t)

    jj = jnp.arange(EMB)
    perm = 32 * (jj // 32) + 2 * (jj % 16) + ((jj % 32) // 16)
    wg1_p = Wg1[perm, :]
    bias_p = bias[:, perm]

    BR = 256
    grid = NTOT // BR
    part_a = pl.pallas_call(
        _aggr_body,
        grid=(grid,),
        in_specs=[pl.BlockSpec((BR, ND), lambda i: (i, 0)),
                  pl.BlockSpec((ND, H1), lambda i: (0, 0))],
        out_specs=pl.BlockSpec((BR, H1), lambda i: (i, 0)),
        out_shape=jax.ShapeDtypeStruct((NTOT, H1), F32),
    )(net_adj, y1d)

    w23 = jnp.concatenate([Wg2, Wg3], axis=1)
    y23 = pl.pallas_call(
        _aggrb_y23_body,
        grid=(grid,),
        in_specs=[pl.BlockSpec((BR, H1), lambda i: (i, 0)),
                  pl.BlockSpec((BR, NPRO), lambda i: (i, ND // NPRO)),
                  pl.BlockSpec((NC_SC, NPRO, EMB), lambda i: (0, 0, 0)),
                  pl.BlockSpec((1, EMB), lambda i: (0, 0)),
                  pl.BlockSpec((EMB, H1), lambda i: (0, 0)),
                  pl.BlockSpec((H1, H1), lambda i: (0, 0))],
        out_specs=pl.BlockSpec((BR, H1), lambda i: (i, 0)),
        out_shape=jax.ShapeDtypeStruct((NTOT, H1), F32),
    )(part_a, net_adj, pro_raw, bias_p, wg1_p, w23)

    mu, logvar = pl.pallas_call(
        _aggr2_body,
        grid=(grid,),
        in_specs=[pl.BlockSpec((BR, NTOT), lambda i: (i, 0)),
                  pl.BlockSpec((NTOT, H1), lambda i: (0, 0))],
        out_specs=[pl.BlockSpec((BR, H2), lambda i: (i, 0)),
                   pl.BlockSpec((BR, H2), lambda i: (i, 0))],
        out_shape=(jax.ShapeDtypeStruct((NTOT, H2), F32),
                   jax.ShapeDtypeStruct((NTOT, H2), F32)),
    )(net_adj, y23)

    BD = 512
    adj_rec = pl.pallas_call(
        _dec_body,
        grid=(NTOT // BD, NTOT // BD),
        in_specs=[pl.BlockSpec((BD, H2), lambda i, j: (i, 0)),
                  pl.BlockSpec((BD, H2), lambda i, j: (j, 0))],
        out_specs=pl.BlockSpec((BD, BD), lambda i, j: (i, j)),
        out_shape=jax.ShapeDtypeStruct((NTOT, NTOT), F32),
    )(mu, mu)

    return adj_rec, mu, logvar


# speedup vs baseline: 1.6295x; 1.6295x over previous
import functools

import jax
import jax.numpy as jnp
from jax import lax
from jax.experimental import pallas as pl
from jax.experimental.pallas import tpu as pltpu
from jax.experimental.pallas import tpu_sc as plsc

F32 = jnp.float32
ND, NPRO, NTOT = 3584, 512, 4096
EMB, H1, H2 = 128, 64, 32
V, L, KW, OC, TT = 26, 1000, 8, 32, 121

NC_SC, NS_SC = 2, 16
NW = NC_SC * NS_SC
G = EMB // 16
LP = 1024
CT = LP // NW
SB = 8
NSUB = CT // SB


def _prep_body(emb_ref, wfc_ref, w5_ref, cbrow_ref, bfc_ref,
               x_ref, w1_ref, b1_ref, w2_ref, b2_ref, w3_ref, b3_ref, wg1_ref,
               u_ref, bias_ref, y1d_ref, t_ref):
    for k in range(KW):
        ek = emb_ref[:, k:k + TT]
        for o in range(OC):
            wo = wfc_ref[pl.ds(o * TT, TT), :]
            t_ref[k, o] = jnp.dot(ek, wo, preferred_element_type=F32)
    for v in range(V):
        tv = t_ref[:, :, v, :].reshape(KW * OC, EMB)
        uv = jnp.dot(w5_ref[...], tv, preferred_element_type=F32)
        ub = uv.astype(jnp.bfloat16).astype(F32)
        lo_i = lax.shift_right_logical(
            lax.bitcast_convert_type(ub[:, :EMB // 2], jnp.int32), 16)
        hi_i = jnp.bitwise_and(
            lax.bitcast_convert_type(ub[:, EMB // 2:], jnp.int32),
            jnp.int32(-65536))
        u_ref[v, pl.ds(0, L), :] = jnp.bitwise_or(lo_i, hi_i)
        u_ref[v, pl.ds(L, LP - L), :] = jnp.zeros((LP - L, EMB // 2), jnp.int32)
    bias_ref[...] = (jnp.dot(cbrow_ref[...], wfc_ref[...], preferred_element_type=F32)
                     + bfc_ref[...])
    h = jnp.maximum(jnp.dot(x_ref[...], w1_ref[...], preferred_element_type=F32)
                    + b1_ref[...], 0.0)
    h = jnp.maximum(jnp.dot(h, w2_ref[...], preferred_element_type=F32)
                    + b2_ref[...], 0.0)
    h = jnp.maximum(jnp.dot(h, w3_ref[...], preferred_element_type=F32)
                    + b3_ref[...], 0.0)
    y1d_ref[...] = jnp.dot(h, wg1_ref[...], preferred_element_type=F32)


def _aggr_body(adj_ref, y_ref, o_ref):
    o_ref[...] = jnp.dot(adj_ref[...], y_ref[...], preferred_element_type=F32)


def _aggrb_y23_body(pa_ref, adj_ref, praw_ref, bias_ref, wg1_ref, w23_ref, o_ref):
    y1p = jnp.dot(praw_ref[0] + praw_ref[1] + bias_ref[...], wg1_ref[...],
                  preferred_element_type=F32)
    h = jnp.maximum(pa_ref[...] + jnp.dot(adj_ref[...], y1p,
                                          preferred_element_type=F32), 0.0)
    o_ref[...] = jnp.dot(h, w23_ref[...], preferred_element_type=F32)


def _aggr2_body(adj_ref, y_ref, mu_ref, lv_ref):
    r = jnp.dot(adj_ref[...], y_ref[...], preferred_element_type=F32)
    mu_ref[...] = r[:, :H2]
    lv_ref[...] = r[:, H2:]


def _dec_body(a_ref, b_ref, o_ref):
    o_ref[...] = lax.dot_general(a_ref[...], b_ref[...],
                                 (((1,), (1,)), ((), ())),
                                 preferred_element_type=F32)


def _sc_body(u_hbm, idx_hbm, out_hbm, u_sub, idx_v, acc_v, idxrow_v, shared):
    cid = lax.axis_index("c")
    sid = lax.axis_index("s")
    wid = sid * NC_SC + cid
    c0 = wid * CT
    pltpu.sync_copy(idx_hbm.at[pl.ds(c0, CT)], idx_v)

    def zbody(n16, carry):
        for g in range(G):
            acc_v[n16, pl.ds(g * 16, 16)] = jnp.zeros((16,), F32)
        return carry

    lax.fori_loop(0, NPRO, zbody, 0)

    def sbody(sub, carry):
        pltpu.sync_copy(u_hbm.at[:, pl.ds(c0 + sub * SB, SB), :], u_sub)

        def nbody(nb, carry2):
            n0 = nb * 16
            iv = [idx_v[sub * SB + c, pl.ds(n0, 16)] for c in range(SB)]
            for j in range(16):
                n = n0 + j
                acc = [acc_v[n, pl.ds(g * 16, 16)] for g in range(G)]
                for c in range(SB):
                    v = iv[c][j]
                    for b4 in range(4):
                        x = u_sub[v, c, pl.ds(b4 * 16, 16)]
                        lo = lax.bitcast_convert_type(lax.shift_left(x, 16), F32)
                        hi = lax.bitcast_convert_type(
                            jnp.bitwise_and(x, jnp.int32(-65536)), F32)
                        acc[b4] = acc[b4] + lo
                        acc[4 + b4] = acc[4 + b4] + hi
                for g in range(G):
                    acc_v[n, pl.ds(g * 16, 16)] = acc[g]
            return carry2

        lax.fori_loop(0, NPRO // 16, nbody, 0)
        return carry

    lax.fori_loop(0, NSUB, sbody, 0)

    for blk in range(NPRO // EMB):
        for g in range(G):
            idxrow_v[blk, pl.ds(g * 16, 16)] = (lax.iota(jnp.int32, 16)
                                                + (blk * EMB + g * 16))

    @pl.when(sid == 0)
    def _():
        pltpu.sync_copy(acc_v, shared)
    plsc.subcore_barrier()

    @pl.when(sid != 0)
    def _():
        for blk in range(NPRO // EMB):
            pltpu.sync_copy(acc_v.at[pl.ds(blk * EMB, EMB)],
                            shared.at[idxrow_v.at[blk]], add=True)
    plsc.subcore_barrier()

    @pl.when(sid == 0)
    def _():
        pltpu.sync_copy(shared, out_hbm.at[cid])


_sc_gather = functools.partial(
    pl.kernel,
    out_type=jax.ShapeDtypeStruct((NC_SC, NPRO, EMB), F32),
    mesh=plsc.VectorSubcoreMesh(core_axis_name="c", subcore_axis_name="s"),
    scratch_types=[
        pltpu.VMEM((V, SB, EMB // 2), jnp.int32),
        pltpu.VMEM((CT, NPRO), jnp.int32),
        pltpu.VMEM((NPRO, EMB), F32),
        pltpu.VMEM((NPRO // EMB, EMB), jnp.int32),
        pltpu.VMEM_SHARED((NPRO, EMB), F32),
    ],
)(_sc_body)


def kernel(drug_x, pro_x, net_adj, W1, b1, W2, b2, W3, b3, emb_table,
           conv_w, conv_b, Wfc, bfc, Wg1, Wg2, Wg3):
    w5 = conv_w.transpose(1, 2, 0).reshape(L, KW * OC)
    cbrow = jnp.repeat(conv_b, TT)[None, :]
    b1r, b2r, b3r, bfcr = b1[None, :], b2[None, :], b3[None, :], bfc[None, :]

    u, bias, y1d = pl.pallas_call(
        _prep_body,
        out_shape=(jax.ShapeDtypeStruct((V, LP, EMB // 2), jnp.int32),
                   jax.ShapeDtypeStruct((1, EMB), F32),
                   jax.ShapeDtypeStruct((ND, H1), F32)),
        scratch_shapes=[pltpu.VMEM((KW, OC, V, EMB), F32)],
    )(emb_table, Wfc, w5, cbrow, bfcr, drug_x, W1, b1r, W2, b2r, W3, b3r, Wg1)

    idx_t = jnp.pad(pro_x.transpose(), ((0, LP - L), (0, 0)))

    pro_raw = _sc_gather(u, idx_t)

    BR = 256
    grid = NTOT // BR
    part_a = pl.pallas_call(
        _aggr_body,
        grid=(grid,),
        in_specs=[pl.BlockSpec((BR, ND), lambda i: (i, 0)),
                  pl.BlockSpec((ND, H1), lambda i: (0, 0))],
        out_specs=pl.BlockSpec((BR, H1), lambda i: (i, 0)),
        out_shape=jax.ShapeDtypeStruct((NTOT, H1), F32),
    )(net_adj, y1d)

    w23 = jnp.concatenate([Wg2, Wg3], axis=1)
    y23 = pl.pallas_call(
        _aggrb_y23_body,
        grid=(grid,),
        in_specs=[pl.BlockSpec((BR, H1), lambda i: (i, 0)),
                  pl.BlockSpec((BR, NPRO), lambda i: (i, ND // NPRO)),
                  pl.BlockSpec((NC_SC, NPRO, EMB), lambda i: (0, 0, 0)),
                  pl.BlockSpec((1, EMB), lambda i: (0, 0)),
                  pl.BlockSpec((EMB, H1), lambda i: (0, 0)),
                  pl.BlockSpec((H1, H1), lambda i: (0, 0))],
        out_specs=pl.BlockSpec((BR, H1), lambda i: (i, 0)),
        out_shape=jax.ShapeDtypeStruct((NTOT, H1), F32),
    )(part_a, net_adj, pro_raw, bias, Wg1, w23)

    mu, logvar = pl.pallas_call(
        _aggr2_body,
        grid=(grid,),
        in_specs=[pl.BlockSpec((BR, NTOT), lambda i: (i, 0)),
                  pl.BlockSpec((NTOT, H1), lambda i: (0, 0))],
        out_specs=[pl.BlockSpec((BR, H2), lambda i: (i, 0)),
                   pl.BlockSpec((BR, H2), lambda i: (i, 0))],
        out_shape=(jax.ShapeDtypeStruct((NTOT, H2), F32),
                   jax.ShapeDtypeStruct((NTOT, H2), F32)),
    )(net_adj, y23)

    BD = 512
    adj_rec = pl.pallas_call(
        _dec_body,
        grid=(NTOT // BD, NTOT // BD),
        in_specs=[pl.BlockSpec((BD, H2), lambda i, j: (i, 0)),
                  pl.BlockSpec((BD, H2), lambda i, j: (j, 0))],
        out_specs=pl.BlockSpec((BD, BD), lambda i, j: (i, j)),
        out_shape=jax.ShapeDtypeStruct((NTOT, NTOT), F32),
    )(mu, mu)

    return adj_rec, mu, logvar


# speedup vs baseline: 1.6685x; 1.0239x over previous
import functools

import jax
import jax.numpy as jnp
from jax import lax
from jax.experimental import pallas as pl
from jax.experimental.pallas import tpu as pltpu
from jax.experimental.pallas import tpu_sc as plsc

F32 = jnp.float32
ND, NPRO, NTOT = 3584, 512, 4096
EMB, H1, H2 = 128, 64, 32
V, L, KW, OC, TT = 26, 1000, 8, 32, 121

NC_SC, NS_SC = 2, 16
NW = NC_SC * NS_SC
G = EMB // 16
LP = 1024
CT = LP // NW
SB = 8
NSUB = CT // SB


def _prep_body(emb_ref, wfc_ref, w5_ref, cbrow_ref, bfc_ref,
               x_ref, w1_ref, b1_ref, w2_ref, b2_ref, w3_ref, b3_ref, wg1_ref,
               u_ref, bias_ref, y1d_ref, t_ref):
    for k in range(KW):
        ek = emb_ref[:, k:k + TT]
        for o in range(OC):
            wo = wfc_ref[pl.ds(o * TT, TT), :]
            t_ref[k, o] = jnp.dot(ek, wo, preferred_element_type=F32)
    for v in range(V):
        tv = t_ref[:, :, v, :].reshape(KW * OC, EMB)
        uv = jnp.dot(w5_ref[...], tv, preferred_element_type=F32)
        ub = uv.astype(jnp.bfloat16).astype(F32)
        lo_i = lax.shift_right_logical(
            lax.bitcast_convert_type(ub[:, :EMB // 2], jnp.int32), 16)
        hi_i = jnp.bitwise_and(
            lax.bitcast_convert_type(ub[:, EMB // 2:], jnp.int32),
            jnp.int32(-65536))
        u_ref[v, pl.ds(0, L), :] = jnp.bitwise_or(lo_i, hi_i)
        u_ref[v, pl.ds(L, LP - L), :] = jnp.zeros((LP - L, EMB // 2), jnp.int32)
    bias_ref[...] = (jnp.dot(cbrow_ref[...], wfc_ref[...], preferred_element_type=F32)
                     + bfc_ref[...])
    h = jnp.maximum(jnp.dot(x_ref[...], w1_ref[...], preferred_element_type=F32)
                    + b1_ref[...], 0.0)
    h = jnp.maximum(jnp.dot(h, w2_ref[...], preferred_element_type=F32)
                    + b2_ref[...], 0.0)
    h = jnp.maximum(jnp.dot(h, w3_ref[...], preferred_element_type=F32)
                    + b3_ref[...], 0.0)
    y1d_ref[...] = jnp.dot(h, wg1_ref[...], preferred_element_type=F32)


BR = 256
BD = 512


def _mega_body(adj_ref, y1d_ref, praw_ref, bias_ref, wg1_ref, w23_ref,
               mu_ref, lv_ref, rec_ref, y23_scr, mu_scr):
    i = pl.program_id(0)

    @pl.when(i < 16)
    def _():
        y1p = jnp.dot(praw_ref[0] + praw_ref[1] + bias_ref[...], wg1_ref[...],
                      preferred_element_type=F32)
        pa = jnp.dot(adj_ref[:, :ND], y1d_ref[...], preferred_element_type=F32)
        pb = jnp.dot(adj_ref[:, ND:], y1p, preferred_element_type=F32)
        h = jnp.maximum(pa + pb, 0.0)
        y23_scr[pl.ds(i * BR, BR), :] = jnp.dot(h, w23_ref[...],
                                                preferred_element_type=F32)

    @pl.when(jnp.logical_and(i >= 16, i < 32))
    def _():
        r = jnp.dot(adj_ref[...], y23_scr[...], preferred_element_type=F32)
        mu_ref[...] = r[:, :H2]
        lv_ref[...] = r[:, H2:]
        mu_scr[pl.ds((i - 16) * BR, BR), :] = r[:, :H2]

    @pl.when(i >= 32)
    def _():
        t = i - 32
        a = mu_scr[pl.ds((t // 8) * BD, BD), :]
        b = mu_scr[pl.ds((t % 8) * BD, BD), :]
        rec_ref[...] = lax.dot_general(a, b, (((1,), (1,)), ((), ())),
                                       preferred_element_type=F32)


def _sc_body(u_hbm, idx_hbm, out_hbm, u_sub, idx_v, acc_v, idxrow_v, shared):
    cid = lax.axis_index("c")
    sid = lax.axis_index("s")
    wid = sid * NC_SC + cid
    c0 = wid * CT
    pltpu.sync_copy(idx_hbm.at[pl.ds(c0, CT)], idx_v)

    def zbody(n16, carry):
        for g in range(G):
            acc_v[n16, pl.ds(g * 16, 16)] = jnp.zeros((16,), F32)
        return carry

    lax.fori_loop(0, NPRO, zbody, 0)

    def sbody(sub, carry):
        pltpu.sync_copy(u_hbm.at[:, pl.ds(c0 + sub * SB, SB), :], u_sub)

        def nbody(nb, carry2):
            n0 = nb * 16
            iv = [idx_v[sub * SB + c, pl.ds(n0, 16)] for c in range(SB)]
            for j in range(16):
                n = n0 + j
                acc = [acc_v[n, pl.ds(g * 16, 16)] for g in range(G)]
                for c in range(SB):
                    v = iv[c][j]
                    for b4 in range(4):
                        x = u_sub[v, c, pl.ds(b4 * 16, 16)]
                        lo = lax.bitcast_convert_type(lax.shift_left(x, 16), F32)
                        hi = lax.bitcast_convert_type(
                            jnp.bitwise_and(x, jnp.int32(-65536)), F32)
                        acc[b4] = acc[b4] + lo
                        acc[4 + b4] = acc[4 + b4] + hi
                for g in range(G):
                    acc_v[n, pl.ds(g * 16, 16)] = acc[g]
            return carry2

        lax.fori_loop(0, NPRO // 16, nbody, 0)
        return carry

    lax.fori_loop(0, NSUB, sbody, 0)

    for blk in range(NPRO // EMB):
        for g in range(G):
            idxrow_v[blk, pl.ds(g * 16, 16)] = (lax.iota(jnp.int32, 16)
                                                + (blk * EMB + g * 16))

    @pl.when(sid == 0)
    def _():
        pltpu.sync_copy(acc_v, shared)
    plsc.subcore_barrier()

    @pl.when(sid != 0)
    def _():
        for blk in range(NPRO // EMB):
            pltpu.sync_copy(acc_v.at[pl.ds(blk * EMB, EMB)],
                            shared.at[idxrow_v.at[blk]], add=True)
    plsc.subcore_barrier()

    @pl.when(sid == 0)
    def _():
        pltpu.sync_copy(shared, out_hbm.at[cid])


_sc_gather = functools.partial(
    pl.kernel,
    out_type=jax.ShapeDtypeStruct((NC_SC, NPRO, EMB), F32),
    mesh=plsc.VectorSubcoreMesh(core_axis_name="c", subcore_axis_name="s"),
    scratch_types=[
        pltpu.VMEM((V, SB, EMB // 2), jnp.int32),
        pltpu.VMEM((CT, NPRO), jnp.int32),
        pltpu.VMEM((NPRO, EMB), F32),
        pltpu.VMEM((NPRO // EMB, EMB), jnp.int32),
        pltpu.VMEM_SHARED((NPRO, EMB), F32),
    ],
)(_sc_body)


def kernel(drug_x, pro_x, net_adj, W1, b1, W2, b2, W3, b3, emb_table,
           conv_w, conv_b, Wfc, bfc, Wg1, Wg2, Wg3):
    w5 = conv_w.transpose(1, 2, 0).reshape(L, KW * OC)
    cbrow = jnp.repeat(conv_b, TT)[None, :]
    b1r, b2r, b3r, bfcr = b1[None, :], b2[None, :], b3[None, :], bfc[None, :]

    u, bias, y1d = pl.pallas_call(
        _prep_body,
        out_shape=(jax.ShapeDtypeStruct((V, LP, EMB // 2), jnp.int32),
                   jax.ShapeDtypeStruct((1, EMB), F32),
                   jax.ShapeDtypeStruct((ND, H1), F32)),
        scratch_shapes=[pltpu.VMEM((KW, OC, V, EMB), F32)],
    )(emb_table, Wfc, w5, cbrow, bfcr, drug_x, W1, b1r, W2, b2r, W3, b3r, Wg1)

    idx_t = jnp.pad(pro_x.transpose(), ((0, LP - L), (0, 0)))

    pro_raw = _sc_gather(u, idx_t)

    w23 = jnp.concatenate([Wg2, Wg3], axis=1)
    adj_row = lambda i: (jnp.where(i < 16, i, jnp.where(i < 32, i - 16, 15)), 0)
    mu_row = lambda i: (jnp.clip(i - 16, 0, 15), 0)
    rec_blk = lambda i: (jnp.where(i < 32, 0, (i - 32) // 8),
                         jnp.where(i < 32, 0, (i - 32) % 8))
    mu, logvar, adj_rec = pl.pallas_call(
        _mega_body,
        grid=(96,),
        in_specs=[pl.BlockSpec((BR, NTOT), adj_row),
                  pl.BlockSpec((ND, H1), lambda i: (0, 0)),
                  pl.BlockSpec((NC_SC, NPRO, EMB), lambda i: (0, 0, 0)),
                  pl.BlockSpec((1, EMB), lambda i: (0, 0)),
                  pl.BlockSpec((EMB, H1), lambda i: (0, 0)),
                  pl.BlockSpec((H1, H1), lambda i: (0, 0))],
        out_specs=[pl.BlockSpec((BR, H2), mu_row),
                   pl.BlockSpec((BR, H2), mu_row),
                   pl.BlockSpec((BD, BD), rec_blk)],
        out_shape=(jax.ShapeDtypeStruct((NTOT, H2), F32),
                   jax.ShapeDtypeStruct((NTOT, H2), F32),
                   jax.ShapeDtypeStruct((NTOT, NTOT), F32)),
        scratch_shapes=[pltpu.VMEM((NTOT, H1), F32),
                        pltpu.VMEM((NTOT, H2), F32)],
    )(net_adj, y1d, pro_raw, bias, Wg1, w23)

    return adj_rec, mu, logvar


# speedup vs baseline: 1.7495x; 1.0486x over previous
import functools

import jax
import jax.numpy as jnp
from jax import lax
from jax.experimental import pallas as pl
from jax.experimental.pallas import tpu as pltpu
from jax.experimental.pallas import tpu_sc as plsc

F32 = jnp.float32
ND, NPRO, NTOT = 3584, 512, 4096
EMB, H1, H2 = 128, 64, 32
V, L, KW, OC, TT = 26, 1000, 8, 32, 121

NC_SC, NS_SC = 2, 16
NW = NC_SC * NS_SC
G = EMB // 16
LP = 1024
CT = LP // NW
SB = 8
NSUB = CT // SB


def _prep_body(emb_ref, wfc_ref, w5_ref, cbrow_ref, bfc_ref,
               x_ref, w1_ref, b1_ref, w2_ref, b2_ref, w3_ref, b3_ref, wg1_ref,
               u_ref, bias_ref, y1d_ref, t_ref):
    for k in range(KW):
        ek = emb_ref[:, k:k + TT]
        for o in range(OC):
            wo = wfc_ref[pl.ds(o * TT, TT), :]
            t_ref[k, o] = jnp.dot(ek, wo, preferred_element_type=F32)
    for v in range(V):
        tv = t_ref[:, :, v, :].reshape(KW * OC, EMB)
        uv = jnp.dot(w5_ref[...], tv, preferred_element_type=F32)
        ub = uv.astype(jnp.bfloat16).astype(F32)
        lo_i = lax.shift_right_logical(
            lax.bitcast_convert_type(ub[:, :EMB // 2], jnp.int32), 16)
        hi_i = jnp.bitwise_and(
            lax.bitcast_convert_type(ub[:, EMB // 2:], jnp.int32),
            jnp.int32(-65536))
        u_ref[v, pl.ds(0, L), :] = jnp.bitwise_or(lo_i, hi_i)
        u_ref[v, pl.ds(L, LP - L), :] = jnp.zeros((LP - L, EMB // 2), jnp.int32)
    bias_ref[...] = (jnp.dot(cbrow_ref[...], wfc_ref[...], preferred_element_type=F32)
                     + bfc_ref[...])
    h = jnp.maximum(jnp.dot(x_ref[...], w1_ref[...], preferred_element_type=F32)
                    + b1_ref[...], 0.0)
    h = jnp.maximum(jnp.dot(h, w2_ref[...], preferred_element_type=F32)
                    + b2_ref[...], 0.0)
    h = jnp.maximum(jnp.dot(h, w3_ref[...], preferred_element_type=F32)
                    + b3_ref[...], 0.0)
    y1d_ref[...] = jnp.dot(h, wg1_ref[...], preferred_element_type=F32)


BR = 256
BD = 512


def _parta_body(adj_ref, y1d_ref, o_ref):
    o_ref[...] = jnp.dot(adj_ref[...], y1d_ref[...], preferred_element_type=F32)


def _mega_body(adjp_ref, pa_ref, adj_ref, praw_ref, bias_ref, wg1_ref, w23_ref,
               mu_ref, lv_ref, rec_ref, y23_scr, mu_scr):
    i = pl.program_id(0)

    @pl.when(i < 16)
    def _():
        y1p = jnp.dot(praw_ref[0] + praw_ref[1] + bias_ref[...], wg1_ref[...],
                      preferred_element_type=F32)
        pb = jnp.dot(adjp_ref[...], y1p, preferred_element_type=F32)
        h = jnp.maximum(pa_ref[...] + pb, 0.0)
        y23_scr[pl.ds(i * BR, BR), :] = jnp.dot(h, w23_ref[...],
                                                preferred_element_type=F32)

    @pl.when(jnp.logical_and(i >= 16, i < 32))
    def _():
        r = jnp.dot(adj_ref[...], y23_scr[...], preferred_element_type=F32)
        mu_ref[...] = r[:, :H2]
        lv_ref[...] = r[:, H2:]
        mu_scr[pl.ds((i - 16) * BR, BR), :] = r[:, :H2]

    @pl.when(i >= 32)
    def _():
        t = i - 32
        a = mu_scr[pl.ds((t // 8) * BD, BD), :]
        b = mu_scr[pl.ds((t % 8) * BD, BD), :]
        rec_ref[...] = lax.dot_general(a, b, (((1,), (1,)), ((), ())),
                                       preferred_element_type=F32)


def _sc_body(u_hbm, idx_hbm, out_hbm, u_sub, idx_v, acc_v, idxrow_v, shared):
    cid = lax.axis_index("c")
    sid = lax.axis_index("s")
    wid = sid * NC_SC + cid
    c0 = wid * CT
    pltpu.sync_copy(idx_hbm.at[pl.ds(c0, CT)], idx_v)

    def zbody(n16, carry):
        for g in range(G):
            acc_v[n16, pl.ds(g * 16, 16)] = jnp.zeros((16,), F32)
        return carry

    lax.fori_loop(0, NPRO, zbody, 0)

    def sbody(sub, carry):
        pltpu.sync_copy(u_hbm.at[:, pl.ds(c0 + sub * SB, SB), :], u_sub)

        def nbody(nb, carry2):
            n0 = nb * 16
            iv = [idx_v[sub * SB + c, pl.ds(n0, 16)] for c in range(SB)]
            for j in range(16):
                n = n0 + j
                acc = [acc_v[n, pl.ds(g * 16, 16)] for g in range(G)]
                for c in range(SB):
                    v = iv[c][j]
                    for b4 in range(4):
                        x = u_sub[v, c, pl.ds(b4 * 16, 16)]
                        lo = lax.bitcast_convert_type(lax.shift_left(x, 16), F32)
                        hi = lax.bitcast_convert_type(
                            jnp.bitwise_and(x, jnp.int32(-65536)), F32)
                        acc[b4] = acc[b4] + lo
                        acc[4 + b4] = acc[4 + b4] + hi
                for g in range(G):
                    acc_v[n, pl.ds(g * 16, 16)] = acc[g]
            return carry2

        lax.fori_loop(0, NPRO // 16, nbody, 0)
        return carry

    lax.fori_loop(0, NSUB, sbody, 0)

    for blk in range(NPRO // EMB):
        for g in range(G):
            idxrow_v[blk, pl.ds(g * 16, 16)] = (lax.iota(jnp.int32, 16)
                                                + (blk * EMB + g * 16))

    @pl.when(sid == 0)
    def _():
        pltpu.sync_copy(acc_v, shared)
    plsc.subcore_barrier()

    @pl.when(sid != 0)
    def _():
        for blk in range(NPRO // EMB):
            pltpu.sync_copy(acc_v.at[pl.ds(blk * EMB, EMB)],
                            shared.at[idxrow_v.at[blk]], add=True)
    plsc.subcore_barrier()

    @pl.when(sid == 0)
    def _():
        pltpu.sync_copy(shared, out_hbm.at[cid])


_sc_gather = functools.partial(
    pl.kernel,
    out_type=jax.ShapeDtypeStruct((NC_SC, NPRO, EMB), F32),
    mesh=plsc.VectorSubcoreMesh(core_axis_name="c", subcore_axis_name="s"),
    scratch_types=[
        pltpu.VMEM((V, SB, EMB // 2), jnp.int32),
        pltpu.VMEM((CT, NPRO), jnp.int32),
        pltpu.VMEM((NPRO, EMB), F32),
        pltpu.VMEM((NPRO // EMB, EMB), jnp.int32),
        pltpu.VMEM_SHARED((NPRO, EMB), F32),
    ],
)(_sc_body)


def kernel(drug_x, pro_x, net_adj, W1, b1, W2, b2, W3, b3, emb_table,
           conv_w, conv_b, Wfc, bfc, Wg1, Wg2, Wg3):
    w5 = conv_w.transpose(1, 2, 0).reshape(L, KW * OC)
    cbrow = jnp.repeat(conv_b, TT)[None, :]
    b1r, b2r, b3r, bfcr = b1[None, :], b2[None, :], b3[None, :], bfc[None, :]

    u, bias, y1d = pl.pallas_call(
        _prep_body,
        out_shape=(jax.ShapeDtypeStruct((V, LP, EMB // 2), jnp.int32),
                   jax.ShapeDtypeStruct((1, EMB), F32),
                   jax.ShapeDtypeStruct((ND, H1), F32)),
        scratch_shapes=[pltpu.VMEM((KW, OC, V, EMB), F32)],
    )(emb_table, Wfc, w5, cbrow, bfcr, drug_x, W1, b1r, W2, b2r, W3, b3r, Wg1)

    idx_t = jnp.pad(pro_x.transpose(), ((0, LP - L), (0, 0)))

    pro_raw = _sc_gather(u, idx_t)

    part_a = pl.pallas_call(
        _parta_body,
        grid=(NTOT // BR,),
        in_specs=[pl.BlockSpec((BR, ND), lambda i: (i, 0)),
                  pl.BlockSpec((ND, H1), lambda i: (0, 0))],
        out_specs=pl.BlockSpec((BR, H1), lambda i: (i, 0)),
        out_shape=jax.ShapeDtypeStruct((NTOT, H1), F32),
    )(net_adj, y1d)

    w23 = jnp.concatenate([Wg2, Wg3], axis=1)
    adjp_row = lambda i: (jnp.clip(i, 0, 15), ND // NPRO)
    adj_row = lambda i: (jnp.where(i < 32, jnp.clip(i - 16, 0, 15), 15), 0)
    mu_row = lambda i: (jnp.clip(i - 16, 0, 15), 0)
    rec_blk = lambda i: (jnp.where(i < 32, 0, (i - 32) // 8),
                         jnp.where(i < 32, 0, (i - 32) % 8))
    mu, logvar, adj_rec = pl.pallas_call(
        _mega_body,
        grid=(96,),
        in_specs=[pl.BlockSpec((BR, NPRO), adjp_row),
                  pl.BlockSpec((BR, H1), lambda i: (jnp.clip(i, 0, 15), 0)),
                  pl.BlockSpec((BR, NTOT), adj_row),
                  pl.BlockSpec((NC_SC, NPRO, EMB), lambda i: (0, 0, 0)),
                  pl.BlockSpec((1, EMB), lambda i: (0, 0)),
                  pl.BlockSpec((EMB, H1), lambda i: (0, 0)),
                  pl.BlockSpec((H1, H1), lambda i: (0, 0))],
        out_specs=[pl.BlockSpec((BR, H2), mu_row),
                   pl.BlockSpec((BR, H2), mu_row),
                   pl.BlockSpec((BD, BD), rec_blk)],
        out_shape=(jax.ShapeDtypeStruct((NTOT, H2), F32),
                   jax.ShapeDtypeStruct((NTOT, H2), F32),
                   jax.ShapeDtypeStruct((NTOT, NTOT), F32)),
        scratch_shapes=[pltpu.VMEM((NTOT, H1), F32),
                        pltpu.VMEM((NTOT, H2), F32)],
    )(net_adj, part_a, net_adj, pro_raw, bias, Wg1, w23)

    return adj_rec, mu, logvar
